# baseline (speedup 1.0000x reference)
import functools

import jax
import jax.numpy as jnp
from jax import lax
from jax.experimental import pallas as pl
from jax.experimental.pallas import tpu as pltpu
from jax.experimental.pallas import tpu_sc as plsc

N = 10000
E = 320000
D = 128
DE = 16
H = 128

NC = 2
NS = 16
NW = NC * NS
LANES = 16

CB = 128
NCHUNK = E // CB
NPAD = 10240
NPT = NPAD // NS

NSLICE = 2
ECH = NCHUNK // NSLICE
ES = ECH * CB


@functools.cache
def _mesh():
    return plsc.VectorSubcoreMesh(core_axis_name="c", subcore_axis_name="s",
                                  num_cores=NC, num_subcores=NS)


def _vadd_rows_into(out, a, b):
    @plsc.parallel_loop(0, CB)
    def _(r):
        for j in range(D // LANES):
            sl = pl.ds(j * LANES, LANES)
            out[r, sl] = a[r, sl] + b[r, sl]


def _make_gather_body(nch):
    nf = nch // NW
    rm = nch % NW
    nh = nf // 2
    odd = nf % 2

    def body(xs_hbm, xt_hbm, row_hbm, col_hbm, out_hbm,
             rowi, coli, a0, b0, a1, b1, g0, g1, tr, tc,
             sem0, sem1, semw0, semw1):
        cid = lax.axis_index("c")
        sid = lax.axis_index("s")
        wid = cid * NS + sid
        sw = wid * nf + jnp.minimum(wid, rm)
        pltpu.sync_copy(row_hbm.at[pl.ds(sw * CB, nf * CB)], rowi)
        pltpu.sync_copy(col_hbm.at[pl.ds(sw * CB, nf * CB)], coli)

        def fire(i, ba, bb, sem):
            pltpu.async_copy(xs_hbm.at[rowi.at[pl.ds(i * CB, CB)]], ba, sem)
            pltpu.async_copy(xt_hbm.at[coli.at[pl.ds(i * CB, CB)]], bb, sem)

        def drain2(ba, bb, sem):
            pltpu.make_async_copy(xs_hbm.at[tr], ba, sem).wait()
            pltpu.make_async_copy(xs_hbm.at[tr], bb, sem).wait()

        def fire_store(i, gb, semw):
            pltpu.async_copy(gb, out_hbm.at[pl.ds((sw + i) * CB, CB)], semw)

        def drain_store(gb, semw):
            pltpu.make_async_copy(gb, out_hbm.at[pl.ds(sw * CB, CB)],
                                  semw).wait()

        fire(0, a0, b0, sem0)

        def step(k, carry):
            i = 2 * k
            fire(i + 1, a1, b1, sem1)
            drain2(a0, b0, sem0)

            @pl.when(k > 0)
            def _():
                drain_store(g0, semw0)

            _vadd_rows_into(g0, a0, b0)
            fire_store(i, g0, semw0)

            @pl.when(2 * k + 2 < nf)
            def _():
                fire(i + 2, a0, b0, sem0)

            drain2(a1, b1, sem1)

            @pl.when(k > 0)
            def _():
                drain_store(g1, semw1)

            _vadd_rows_into(g1, a1, b1)
            fire_store(i + 1, g1, semw1)
            return carry

        lax.fori_loop(0, nh, step, 0)

        if odd:
            drain2(a0, b0, sem0)
            drain_store(g0, semw0)
            _vadd_rows_into(g0, a0, b0)
            fire_store(nf - 1, g0, semw0)
        drain_store(g0, semw0)
        drain_store(g1, semw1)

        @pl.when(wid < rm)
        def _():
            base = (sw + nf) * CB
            pltpu.sync_copy(row_hbm.at[pl.ds(base, CB)], tr)
            pltpu.sync_copy(col_hbm.at[pl.ds(base, CB)], tc)
            pltpu.async_copy(xs_hbm.at[tr], a0, sem0)
            pltpu.async_copy(xt_hbm.at[tc], b0, sem0)
            drain2(a0, b0, sem0)
            _vadd_rows_into(g0, a0, b0)
            pltpu.sync_copy(g0, out_hbm.at[pl.ds(base, CB)])

    return body, nf


def _make_gather(nch):
    body, nf = _make_gather_body(nch)

    @jax.jit
    def run(xs, xt, row, col):
        return pl.kernel(
            body,
            out_type=jax.ShapeDtypeStruct((nch * CB, D), jnp.float32),
            mesh=_mesh(),
            scratch_types=[
                pltpu.VMEM((nf * CB,), jnp.int32),
                pltpu.VMEM((nf * CB,), jnp.int32),
                pltpu.VMEM((CB, D), jnp.float32),
                pltpu.VMEM((CB, D), jnp.float32),
                pltpu.VMEM((CB, D), jnp.float32),
                pltpu.VMEM((CB, D), jnp.float32),
                pltpu.VMEM((CB, D), jnp.float32),
                pltpu.VMEM((CB, D), jnp.float32),
                pltpu.VMEM((CB,), jnp.int32),
                pltpu.VMEM((CB,), jnp.int32),
                pltpu.SemaphoreType.DMA,
                pltpu.SemaphoreType.DMA,
                pltpu.SemaphoreType.DMA,
                pltpu.SemaphoreType.DMA,
            ],
        )(xs, xt, row, col)

    return run


_gather_slice = _make_gather(ECH)


def _zero_rows(buf):
    @plsc.parallel_loop(0, CB)
    def _(r):
        for j in range(D // LANES):
            buf[r, pl.ds(j * LANES, LANES)] = jnp.zeros((LANES,), jnp.float32)


def _make_scatter_body(nch, choff):
    shalf = nch // NC
    sf = shalf // NS
    sr = shalf % NS
    sh2 = sf // 2
    sodd = sf % 2

    def body(ef_hbm, row_hbm, out_hbm, r0, r1, e0, e1, sem0, sem1, agg_sh):
        cid = lax.axis_index("c")
        sid = lax.axis_index("s")
        _zero_rows(e0)
        for k in range(NPT // CB):
            pltpu.sync_copy(e0, agg_sh.at[pl.ds(sid * NPT + k * CB, CB)])
        plsc.subcore_barrier()

        sw = cid * shalf + sid * sf + jnp.minimum(sid, sr)

        def fire(i, eb, rb, sem):
            pltpu.async_copy(
                row_hbm.at[pl.ds((choff + sw + i) * CB, CB)], rb, sem)
            pltpu.async_copy(ef_hbm.at[pl.ds((sw + i) * CB, CB)], eb, sem)

        def drain(eb, rb, sem):
            pltpu.make_async_copy(row_hbm.at[pl.ds(0, CB)], rb, sem).wait()
            pltpu.make_async_copy(ef_hbm.at[pl.ds(0, CB)], eb, sem).wait()

        def scat(eb, rb):
            pltpu.sync_copy(eb, agg_sh.at[rb], add=True)

        fire(0, e0, r0, sem0)

        def step(k, c):
            i = 2 * k
            fire(i + 1, e1, r1, sem1)
            drain(e0, r0, sem0)
            scat(e0, r0)

            @pl.when(2 * k + 2 < sf)
            def _():
                fire(i + 2, e0, r0, sem0)

            drain(e1, r1, sem1)
            scat(e1, r1)
            return c

        lax.fori_loop(0, sh2, step, 0)

        if sodd:
            drain(e0, r0, sem0)
            scat(e0, r0)

        @pl.when(sid < sr)
        def _():
            base = (sw + sf) * CB
            pltpu.sync_copy(row_hbm.at[pl.ds((choff * CB) + base, CB)], r0)
            pltpu.sync_copy(ef_hbm.at[pl.ds(base, CB)], e0)
            pltpu.sync_copy(e0, agg_sh.at[r0], add=True)

        plsc.subcore_barrier()
        for k in range(NPT // CB):
            pltpu.sync_copy(agg_sh.at[pl.ds(sid * NPT + k * CB, CB)], e0)
            pltpu.sync_copy(e0, out_hbm.at[cid, pl.ds(sid * NPT + k * CB, CB)])

    return body


def _make_scatter(nch, choff):
    body = _make_scatter_body(nch, choff)

    @jax.jit
    def run(ef, row):
        return pl.kernel(
            body,
            out_type=jax.ShapeDtypeStruct((NC, NPAD, D), jnp.float32),
            mesh=_mesh(),
            scratch_types=[
                pltpu.VMEM((CB,), jnp.int32),
                pltpu.VMEM((CB,), jnp.int32),
                pltpu.VMEM((CB, D), jnp.float32),
                pltpu.VMEM((CB, D), jnp.float32),
                pltpu.SemaphoreType.DMA,
                pltpu.SemaphoreType.DMA,
                pltpu.VMEM_SHARED((NPAD, D), jnp.float32),
            ],
        )(ef, row)

    return run


_scatter_s0 = _make_scatter(ECH, 0)
_scatter_s1 = _make_scatter(ECH, ECH)


def _proj_body(x_ref, ws_ref, wt_ref, xs_ref, xt_ref):
    x = x_ref[...]
    xs_ref[...] = jnp.dot(x, ws_ref[...], preferred_element_type=jnp.float32)
    xt_ref[...] = jnp.dot(x, wt_ref[...], preferred_element_type=jnp.float32)


def _proj(x, ws, wt, bn):
    nblk = N // bn
    return pl.pallas_call(
        _proj_body,
        grid=(nblk,),
        in_specs=[
            pl.BlockSpec((bn, D), lambda i: (i, 0)),
            pl.BlockSpec((D, H), lambda i: (0, 0)),
            pl.BlockSpec((D, H), lambda i: (0, 0)),
        ],
        out_specs=[
            pl.BlockSpec((bn, H), lambda i: (i, 0)),
            pl.BlockSpec((bn, H), lambda i: (i, 0)),
        ],
        out_shape=[
            jax.ShapeDtypeStruct((N, H), jnp.float32),
            jax.ShapeDtypeStruct((N, H), jnp.float32),
        ],
    )(x, ws, wt)


def _edge_mlp_body(g_ref, ea_ref, we_ref, be1_ref, w2_ref, be2_ref, ef_ref):
    ea1 = jax.lax.dot_general(ea_ref[...], we_ref[...],
                              (((0,), (0,)), ((), ())),
                              preferred_element_type=jnp.float32)
    eh = jnp.maximum(g_ref[...] + ea1 + be1_ref[...], 0.0)
    ef = jnp.dot(eh, w2_ref[...], preferred_element_type=jnp.float32)
    ef_ref[...] = jnp.maximum(ef + be2_ref[...], 0.0)


def _make_edge_mlp(eaoff, be):
    nblk = ES // be
    return pl.pallas_call(
        _edge_mlp_body,
        grid=(nblk,),
        in_specs=[
            pl.BlockSpec((be, H), lambda i: (i, 0)),
            pl.BlockSpec((DE, be), lambda i: (0, i + eaoff)),
            pl.BlockSpec((DE, H), lambda i: (0, 0)),
            pl.BlockSpec((1, H), lambda i: (0, 0)),
            pl.BlockSpec((H, H), lambda i: (0, 0)),
            pl.BlockSpec((1, H), lambda i: (0, 0)),
        ],
        out_specs=pl.BlockSpec((be, H), lambda i: (i, 0)),
        out_shape=jax.ShapeDtypeStruct((ES, H), jnp.float32),
    )


def _node_mlp_body(x_ref, p_ref, q_ref, w1x_ref, w1a_ref, bn1_ref, w2_ref,
                   bn2_ref, out_ref):
    x = x_ref[...]
    agg = p_ref[0] + p_ref[1] + q_ref[0] + q_ref[1]
    nh = jnp.dot(x, w1x_ref[...], preferred_element_type=jnp.float32)
    nh = nh + jnp.dot(agg, w1a_ref[...], preferred_element_type=jnp.float32)
    nh = jnp.maximum(nh + bn1_ref[...], 0.0)
    out_ref[...] = (jnp.dot(nh, w2_ref[...], preferred_element_type=jnp.float32)
                    + bn2_ref[...] + x)


def _node_mlp(x, p, q, w1x, w1a, bn1, w2, bn2, bn):
    nblk = N // bn
    return pl.pallas_call(
        _node_mlp_body,
        grid=(nblk,),
        in_specs=[
            pl.BlockSpec((bn, D), lambda i: (i, 0)),
            pl.BlockSpec((NC, bn, H), lambda i: (0, i, 0)),
            pl.BlockSpec((NC, bn, H), lambda i: (0, i, 0)),
            pl.BlockSpec((D, H), lambda i: (0, 0)),
            pl.BlockSpec((H, H), lambda i: (0, 0)),
            pl.BlockSpec((1, H), lambda i: (0, 0)),
            pl.BlockSpec((H, D), lambda i: (0, 0)),
            pl.BlockSpec((1, D), lambda i: (0, 0)),
        ],
        out_specs=pl.BlockSpec((bn, D), lambda i: (i, 0)),
        out_shape=jax.ShapeDtypeStruct((N, D), jnp.float32),
    )(x, p, q, w1x, w1a, bn1, w2, bn2)


def kernel(x, edge_index, edge_attr, We1, be1, We2, be2, Wn1, bn1, Wn2, bn2):
    row = edge_index[0]
    col = edge_index[1]
    row1 = lax.slice(row, (0,), (ES,))
    col1 = lax.slice(col, (0,), (ES,))
    row2 = lax.slice(row, (ES,), (E,))
    col2 = lax.slice(col, (ES,), (E,))
    xs, xt = _proj(x, We1[:D], We1[D:2 * D], 2000)
    g1 = _gather_slice(xs, xt, row1, col1)
    g2 = _gather_slice(xs, xt, row2, col2)
    we = We1[2 * D:]
    b1 = be1.reshape(1, H)
    b2 = be2.reshape(1, H)
    ea_t = edge_attr.T
    ef1 = _make_edge_mlp(0, 3200)(g1, ea_t, we, b1, We2, b2)
    ef2 = _make_edge_mlp(ES // 3200, 3200)(g2, ea_t, we, b1, We2, b2)
    p = _scatter_s0(ef1, row)
    q = _scatter_s1(ef2, row)
    x_out = _node_mlp(x, p, q, Wn1[:D], Wn1[D:], bn1.reshape(1, H),
                      Wn2, bn2.reshape(1, D), 2000)
    ef = jnp.concatenate([ef1, ef2], axis=0)
    return (x_out, ef)


# speedup vs baseline: 1.0331x; 1.0331x over previous
import functools

import jax
import jax.numpy as jnp
from jax import lax
from jax.experimental import pallas as pl
from jax.experimental.pallas import tpu as pltpu
from jax.experimental.pallas import tpu_sc as plsc

N = 10000
E = 320000
D = 128
DE = 16
H = 128

NC = 2
NS = 16
NW = NC * NS
LANES = 16

CB = 128
NCHUNK = E // CB
NPAD = 10240
NPT = NPAD // NS

@functools.cache
def _mesh():
    return plsc.VectorSubcoreMesh(core_axis_name="c", subcore_axis_name="s",
                                  num_cores=NC, num_subcores=NS)


NFULL = NCHUNK // NW
REM = NCHUNK % NW
NH = NFULL // 2


def _vadd_rows_into(out, a, b):
    @plsc.parallel_loop(0, CB)
    def _(r):
        for j in range(D // LANES):
            sl = pl.ds(j * LANES, LANES)
            out[r, sl] = a[r, sl] + b[r, sl]


def _gather_sum_body(xs_hbm, xt_hbm, row_hbm, col_hbm, out_hbm,
                     rowi, coli, a0, b0, a1, b1, g0, g1, tr, tc,
                     sem0, sem1, semw0, semw1):
    cid = lax.axis_index("c")
    sid = lax.axis_index("s")
    wid = cid * NS + sid
    sw = wid * NFULL + jnp.minimum(wid, REM)
    pltpu.sync_copy(row_hbm.at[pl.ds(sw * CB, NFULL * CB)], rowi)
    pltpu.sync_copy(col_hbm.at[pl.ds(sw * CB, NFULL * CB)], coli)

    def fire(i, ba, bb, sem):
        pltpu.async_copy(xs_hbm.at[rowi.at[pl.ds(i * CB, CB)]], ba, sem)
        pltpu.async_copy(xt_hbm.at[coli.at[pl.ds(i * CB, CB)]], bb, sem)

    def drain2(ba, bb, sem):
        pltpu.make_async_copy(xs_hbm.at[tr], ba, sem).wait()
        pltpu.make_async_copy(xs_hbm.at[tr], bb, sem).wait()

    def fire_store(i, gb, semw):
        pltpu.async_copy(gb, out_hbm.at[pl.ds((sw + i) * CB, CB)], semw)

    def drain_store(gb, semw):
        pltpu.make_async_copy(gb, out_hbm.at[pl.ds(sw * CB, CB)], semw).wait()

    fire(0, a0, b0, sem0)

    def step(k, carry):
        i = 2 * k
        fire(i + 1, a1, b1, sem1)
        drain2(a0, b0, sem0)

        @pl.when(k > 0)
        def _():
            drain_store(g0, semw0)

        _vadd_rows_into(g0, a0, b0)
        fire_store(i, g0, semw0)

        @pl.when(k < NH - 1)
        def _():
            fire(i + 2, a0, b0, sem0)

        drain2(a1, b1, sem1)

        @pl.when(k > 0)
        def _():
            drain_store(g1, semw1)

        _vadd_rows_into(g1, a1, b1)
        fire_store(i + 1, g1, semw1)
        return carry

    lax.fori_loop(0, NH, step, 0)
    drain_store(g0, semw0)
    drain_store(g1, semw1)

    @pl.when(wid < REM)
    def _():
        base = (sw + NFULL) * CB
        pltpu.sync_copy(row_hbm.at[pl.ds(base, CB)], tr)
        pltpu.sync_copy(col_hbm.at[pl.ds(base, CB)], tc)
        pltpu.async_copy(xs_hbm.at[tr], a0, sem0)
        pltpu.async_copy(xt_hbm.at[tc], b0, sem0)
        drain2(a0, b0, sem0)
        _vadd_rows_into(g0, a0, b0)
        pltpu.sync_copy(g0, out_hbm.at[pl.ds(base, CB)])


@jax.jit
def _gather_sum(xs, xt, row, col):
    return pl.kernel(
        _gather_sum_body,
        out_type=jax.ShapeDtypeStruct((E, D), jnp.float32),
        mesh=_mesh(),
        scratch_types=[
            pltpu.VMEM((NFULL * CB,), jnp.int32),
            pltpu.VMEM((NFULL * CB,), jnp.int32),
            pltpu.VMEM((CB, D), jnp.float32),
            pltpu.VMEM((CB, D), jnp.float32),
            pltpu.VMEM((CB, D), jnp.float32),
            pltpu.VMEM((CB, D), jnp.float32),
            pltpu.VMEM((CB, D), jnp.float32),
            pltpu.VMEM((CB, D), jnp.float32),
            pltpu.VMEM((CB,), jnp.int32),
            pltpu.VMEM((CB,), jnp.int32),
            pltpu.SemaphoreType.DMA,
            pltpu.SemaphoreType.DMA,
            pltpu.SemaphoreType.DMA,
            pltpu.SemaphoreType.DMA,
        ],
    )(xs, xt, row, col)


SHALF = NCHUNK // NC
SFULL = SHALF // NS
SREM = SHALF % NS
SH2 = SFULL // 2


def _zero_rows(buf):
    @plsc.parallel_loop(0, CB)
    def _(r):
        for j in range(D // LANES):
            buf[r, pl.ds(j * LANES, LANES)] = jnp.zeros((LANES,), jnp.float32)


def _scatter_add_body(ef_hbm, row_hbm, out_hbm, r0, r1, e0, e1, sem0, sem1,
                      agg_sh):
    cid = lax.axis_index("c")
    sid = lax.axis_index("s")
    _zero_rows(e0)
    for k in range(NPT // CB):
        pltpu.sync_copy(e0, agg_sh.at[pl.ds(sid * NPT + k * CB, CB)])
    plsc.subcore_barrier()

    sw = cid * SHALF + sid * SFULL + jnp.minimum(sid, SREM)

    def fire(i, eb, rb, sem):
        base = (sw + i) * CB
        pltpu.async_copy(row_hbm.at[pl.ds(base, CB)], rb, sem)
        pltpu.async_copy(ef_hbm.at[pl.ds(base, CB)], eb, sem)

    def drain(eb, rb, sem):
        pltpu.make_async_copy(row_hbm.at[pl.ds(0, CB)], rb, sem).wait()
        pltpu.make_async_copy(ef_hbm.at[pl.ds(0, CB)], eb, sem).wait()

    def scat(eb, rb):
        pltpu.sync_copy(eb, agg_sh.at[rb], add=True)

    fire(0, e0, r0, sem0)

    def step(k, c):
        i = 2 * k
        fire(i + 1, e1, r1, sem1)
        drain(e0, r0, sem0)
        scat(e0, r0)

        @pl.when(k < SH2 - 1)
        def _():
            fire(i + 2, e0, r0, sem0)

        drain(e1, r1, sem1)
        scat(e1, r1)
        return c

    lax.fori_loop(0, SH2, step, 0)

    @pl.when(sid < SREM)
    def _():
        base = (sw + SFULL) * CB
        pltpu.sync_copy(row_hbm.at[pl.ds(base, CB)], r0)
        pltpu.sync_copy(ef_hbm.at[pl.ds(base, CB)], e0)
        pltpu.sync_copy(e0, agg_sh.at[r0], add=True)

    plsc.subcore_barrier()
    for k in range(NPT // CB):
        pltpu.sync_copy(agg_sh.at[pl.ds(sid * NPT + k * CB, CB)], e0)
        pltpu.sync_copy(e0, out_hbm.at[cid, pl.ds(sid * NPT + k * CB, CB)])


@jax.jit
def _scatter_add(ef, row):
    return pl.kernel(
        _scatter_add_body,
        out_type=jax.ShapeDtypeStruct((NC, NPAD, D), jnp.float32),
        mesh=_mesh(),
        scratch_types=[
            pltpu.VMEM((CB,), jnp.int32),
            pltpu.VMEM((CB,), jnp.int32),
            pltpu.VMEM((CB, D), jnp.float32),
            pltpu.VMEM((CB, D), jnp.float32),
            pltpu.SemaphoreType.DMA,
            pltpu.SemaphoreType.DMA,
            pltpu.VMEM_SHARED((NPAD, D), jnp.float32),
        ],
    )(ef, row)


def _proj_body(x_ref, ws_ref, wt_ref, xs_ref, xt_ref):
    x = x_ref[...]
    xs_ref[...] = jnp.dot(x, ws_ref[...], preferred_element_type=jnp.float32)
    xt_ref[...] = jnp.dot(x, wt_ref[...], preferred_element_type=jnp.float32)


def _proj(x, ws, wt, bn):
    nblk = N // bn
    return pl.pallas_call(
        _proj_body,
        grid=(nblk,),
        in_specs=[
            pl.BlockSpec((bn, D), lambda i: (i, 0)),
            pl.BlockSpec((D, H), lambda i: (0, 0)),
            pl.BlockSpec((D, H), lambda i: (0, 0)),
        ],
        out_specs=[
            pl.BlockSpec((bn, H), lambda i: (i, 0)),
            pl.BlockSpec((bn, H), lambda i: (i, 0)),
        ],
        out_shape=[
            jax.ShapeDtypeStruct((N, H), jnp.float32),
            jax.ShapeDtypeStruct((N, H), jnp.float32),
        ],
    )(x, ws, wt)


def _edge_mlp_body(g_ref, ea_ref, we_ref, be1_ref, w2_ref, be2_ref, out_ref):
    ea1 = jax.lax.dot_general(ea_ref[...], we_ref[...],
                              (((0,), (0,)), ((), ())),
                              preferred_element_type=jnp.float32)
    eh = jnp.maximum(g_ref[...] + ea1 + be1_ref[...], 0.0)
    ef = jnp.dot(eh, w2_ref[...], preferred_element_type=jnp.float32)
    out_ref[...] = jnp.maximum(ef + be2_ref[...], 0.0)


def _edge_mlp(g, ea, we, be1, w2, be2, be):
    nblk = E // be
    return pl.pallas_call(
        _edge_mlp_body,
        grid=(nblk,),
        in_specs=[
            pl.BlockSpec((be, H), lambda i: (i, 0)),
            pl.BlockSpec((DE, be), lambda i: (0, i)),
            pl.BlockSpec((DE, H), lambda i: (0, 0)),
            pl.BlockSpec((1, H), lambda i: (0, 0)),
            pl.BlockSpec((H, H), lambda i: (0, 0)),
            pl.BlockSpec((1, H), lambda i: (0, 0)),
        ],
        out_specs=pl.BlockSpec((be, H), lambda i: (i, 0)),
        out_shape=jax.ShapeDtypeStruct((E, H), jnp.float32),
    )(g, ea, we, be1, w2, be2)


def _node_mlp_body(x_ref, p_ref, w1x_ref, w1a_ref, bn1_ref, w2_ref, bn2_ref,
                   out_ref):
    x = x_ref[...]
    agg = p_ref[0] + p_ref[1]
    nh = jnp.dot(x, w1x_ref[...], preferred_element_type=jnp.float32)
    nh = nh + jnp.dot(agg, w1a_ref[...], preferred_element_type=jnp.float32)
    nh = jnp.maximum(nh + bn1_ref[...], 0.0)
    out_ref[...] = (jnp.dot(nh, w2_ref[...], preferred_element_type=jnp.float32)
                    + bn2_ref[...] + x)


def _node_mlp(x, p, w1x, w1a, bn1, w2, bn2, bn):
    nblk = N // bn
    return pl.pallas_call(
        _node_mlp_body,
        grid=(nblk,),
        in_specs=[
            pl.BlockSpec((bn, D), lambda i: (i, 0)),
            pl.BlockSpec((NC, bn, H), lambda i: (0, i, 0)),
            pl.BlockSpec((D, H), lambda i: (0, 0)),
            pl.BlockSpec((H, H), lambda i: (0, 0)),
            pl.BlockSpec((1, H), lambda i: (0, 0)),
            pl.BlockSpec((H, D), lambda i: (0, 0)),
            pl.BlockSpec((1, D), lambda i: (0, 0)),
        ],
        out_specs=pl.BlockSpec((bn, D), lambda i: (i, 0)),
        out_shape=jax.ShapeDtypeStruct((N, D), jnp.float32),
    )(x, p, w1x, w1a, bn1, w2, bn2)


def kernel(x, edge_index, edge_attr, We1, be1, We2, be2, Wn1, bn1, Wn2, bn2):
    row = edge_index[0]
    col = edge_index[1]
    xs, xt = _proj(x, We1[:D], We1[D:2 * D], 2000)
    g = _gather_sum(xs, xt, row, col)
    ef = _edge_mlp(g, edge_attr.T, We1[2 * D:], be1.reshape(1, H),
                   We2, be2.reshape(1, H), 2560)
    p = _scatter_add(ef, row)
    x_out = _node_mlp(x, p, Wn1[:D], Wn1[D:], bn1.reshape(1, H),
                      Wn2, bn2.reshape(1, D), 2000)
    return (x_out, ef)
